# baseline (speedup 1.0000x reference)
import jax
import jax.numpy as jnp
from jax.experimental import pallas as pl
from jax.experimental.pallas import tpu as pltpu

N = 4096
D = 256
BM = 256
NB = N // BM
CHUNK = 1024


def _fused_gcn_kernel(adj_ref, x_ref, w1_ref, b1_ref, w2_ref, b2_ref,
                      o_ref, adjbf_ref, s2_ref):
    m = pl.program_id(0)

    for c in range(2, NB + 2):
        @pl.when(m == c)
        def _(c=c):
            b = c - 2
            r0, r1 = b * BM, (b + 1) * BM
            o_ref[r0:r1, :] = jnp.broadcast_to(b2_ref[...], (BM, D)) + jnp.dot(
                adjbf_ref[r0:r1, :r1], s2_ref[:r1, :],
                preferred_element_type=jnp.float32,
            )
            s2_b = s2_ref[r0:r1, :]
            for q0 in range(0, r0, CHUNK):
                q1 = min(q0 + CHUNK, r0)
                o_ref[q0:q1, :] += jnp.dot(
                    adjbf_ref[q0:q1, r0:r1], s2_b,
                    preferred_element_type=jnp.float32,
                )

    @pl.when(jnp.logical_and(m >= 1, m <= NB))
    def _():
        b0 = (m - 1) * BM
        arow = adjbf_ref[pl.ds(b0, BM), :]
        u = jnp.dot(arow, x_ref[...], preferred_element_type=jnp.float32)
        t = jnp.dot(
            u.astype(jnp.bfloat16), w1_ref[...],
            preferred_element_type=jnp.float32,
        )
        h = jnp.maximum(t + b1_ref[...], 0.0).astype(jnp.bfloat16)
        s2_ref[pl.ds(b0, BM), :] = jnp.dot(
            h, w2_ref[...], preferred_element_type=jnp.float32
        ).astype(jnp.bfloat16)

    @pl.when(m < NB)
    def _():
        adjbf_ref[pl.ds(m * BM, BM), :] = adj_ref[...].astype(jnp.bfloat16)


def kernel(x, adj, W1, b1, W2, b2):
    xb = x.astype(jnp.bfloat16)
    w1b = W1.astype(jnp.bfloat16)
    w2b = W2.astype(jnp.bfloat16)
    b1r = b1.reshape(1, D)
    b2r = b2.reshape(1, D)
    return pl.pallas_call(
        _fused_gcn_kernel,
        grid=(NB + 2,),
        in_specs=[
            pl.BlockSpec((BM, N), lambda i: (jnp.minimum(i, NB - 1), 0)),
            pl.BlockSpec((N, D), lambda i: (0, 0)),
            pl.BlockSpec((D, D), lambda i: (0, 0)),
            pl.BlockSpec((1, D), lambda i: (0, 0)),
            pl.BlockSpec((D, D), lambda i: (0, 0)),
            pl.BlockSpec((1, D), lambda i: (0, 0)),
        ],
        out_specs=pl.BlockSpec((N, D), lambda i: (0, 0)),
        out_shape=jax.ShapeDtypeStruct((N, D), jnp.float32),
        scratch_shapes=[
            pltpu.VMEM((N, N), jnp.bfloat16),
            pltpu.VMEM((N, D), jnp.bfloat16),
        ],
    )(adj, xb, w1b, b1r, w2b, b2r)
